# Initial kernel scaffold; baseline (speedup 1.0000x reference)
import jax
import jax.numpy as jnp
from jax.experimental import pallas as pl


def kernel(fs, faces):
    raise NotImplementedError("write your pallas kernel here")


# speedup vs baseline: 57.8434x; 57.8434x over previous
import functools

import jax
import jax.numpy as jnp
from jax import lax
from jax.experimental import pallas as pl
from jax.experimental.pallas import tpu as pltpu
from jax.experimental.pallas import tpu_sc as plsc

NC = 2
NS = 16
L = 16
NW = NC * NS
BLK = 128

_PI = 3.14159265358979


def _rsqrt(x):
    i = lax.bitcast_convert_type(x, jnp.int32)
    i = jnp.int32(0x5F3759DF) - jnp.right_shift(i, 1)
    y = lax.bitcast_convert_type(i, jnp.float32)
    xh = 0.5 * x
    y = y * (1.5 - xh * y * y)
    y = y * (1.5 - xh * y * y)
    y = y * (1.5 - xh * y * y)
    return y


def _acos(x):
    ax = jnp.abs(x)
    u = 1.0 - ax
    s = u * _rsqrt(jnp.maximum(u, 1e-30))
    p = jnp.float32(-0.0012624911)
    p = p * ax + 0.0066700901
    p = p * ax + -0.0170881256
    p = p * ax + 0.0308918810
    p = p * ax + -0.0501743046
    p = p * ax + 0.0889789874
    p = p * ax + -0.2145988016
    p = p * ax + 1.5707963050
    r = s * p
    return jnp.where(x < 0, _PI - r, r)


def _build_sc_call(V, F_PAD, D):
    FPW = F_PAD // NW
    NBLK = FPW // BLK
    mesh = plsc.VectorSubcoreMesh(core_axis_name="c", subcore_axis_name="s")

    @functools.partial(
        pl.kernel,
        out_type=jax.ShapeDtypeStruct((F_PAD, D), jnp.float32),
        mesh=mesh,
        compiler_params=pltpu.CompilerParams(use_tc_tiling_on_sc=False),
        scratch_types=[
            pltpu.VMEM((BLK,), jnp.int32),
            pltpu.VMEM((BLK,), jnp.int32),
            pltpu.VMEM((BLK,), jnp.int32),
            pltpu.VMEM((BLK, D), jnp.float32),
            pltpu.VMEM((BLK, D), jnp.float32),
            pltpu.VMEM((BLK, D), jnp.float32),
            pltpu.VMEM((BLK, D), jnp.float32),
            pltpu.SemaphoreType.DMA,
        ],
    )
    def sc_angles(fs_hbm, i0_hbm, i1_hbm, i2_hbm, out_hbm,
                  i0_v, i1_v, i2_v, p0_v, p1_v, p2_v, o_v, sem):
        wid = lax.axis_index("s") * NC + lax.axis_index("c")
        base = wid * FPW

        @pl.loop(0, NBLK)
        def _(blk):
            off = base + blk * BLK
            pltpu.sync_copy(i0_hbm.at[pl.ds(off, BLK)], i0_v)
            pltpu.sync_copy(i1_hbm.at[pl.ds(off, BLK)], i1_v)
            pltpu.sync_copy(i2_hbm.at[pl.ds(off, BLK)], i2_v)
            cp0 = pltpu.async_copy(fs_hbm.at[i0_v], p0_v, sem)
            cp1 = pltpu.async_copy(fs_hbm.at[i1_v], p1_v, sem)
            cp2 = pltpu.async_copy(fs_hbm.at[i2_v], p2_v, sem)
            cp0.wait()
            cp1.wait()
            cp2.wait()

            @pl.loop(0, BLK)
            def _(f):
                p0x = p0_v[f, pl.ds(0, L)]
                p0y = p0_v[f, pl.ds(L, L)]
                p0z = p0_v[f, pl.ds(2 * L, L)]
                p1x = p1_v[f, pl.ds(0, L)]
                p1y = p1_v[f, pl.ds(L, L)]
                p1z = p1_v[f, pl.ds(2 * L, L)]
                p2x = p2_v[f, pl.ds(0, L)]
                p2y = p2_v[f, pl.ds(L, L)]
                p2z = p2_v[f, pl.ds(2 * L, L)]
                e0x, e0y, e0z = p2x - p0x, p2y - p0y, p2z - p0z
                e1x, e1y, e1z = p0x - p1x, p0y - p1y, p0z - p1z
                e2x, e2y, e2z = p1x - p2x, p1y - p2y, p1z - p2z
                n0 = e0x * e0x + e0y * e0y + e0z * e0z
                n1 = e1x * e1x + e1y * e1y + e1z * e1z
                n2 = e2x * e2x + e2y * e2y + e2z * e2z
                m12 = e1x * e2x + e1y * e2y + e1z * e2z
                m20 = e2x * e0x + e2y * e0y + e2z * e0z
                m01 = e0x * e1x + e0y * e1y + e0z * e1z
                c0 = -m12 * _rsqrt(n1 * n2)
                c1 = -m20 * _rsqrt(n2 * n0)
                c2 = -m01 * _rsqrt(n0 * n1)
                one = jnp.float32(1.0)
                c0 = jnp.minimum(jnp.maximum(c0, -one), one)
                c1 = jnp.minimum(jnp.maximum(c1, -one), one)
                c2 = jnp.minimum(jnp.maximum(c2, -one), one)
                o_v[f, pl.ds(0, L)] = _acos(c0)
                o_v[f, pl.ds(L, L)] = _acos(c1)
                o_v[f, pl.ds(2 * L, L)] = _acos(c2)

            pltpu.sync_copy(o_v, out_hbm.at[pl.ds(off, BLK)])

    return sc_angles


def kernel(fs, faces):
    B, V, _ = fs.shape
    F = faces.shape[0]
    assert B == L
    D = 3 * B
    FPW = -(-F // (NW * BLK)) * BLK
    F_PAD = FPW * NW

    fs_t = fs.transpose(1, 2, 0).reshape(V, D)
    faces_pad = jnp.pad(faces, ((0, F_PAD - F), (0, 0)))
    i0 = faces_pad[:, 0]
    i1 = faces_pad[:, 1]
    i2 = faces_pad[:, 2]

    out = _build_sc_call(V, F_PAD, D)(fs_t, i0, i1, i2)
    alphas = out[:F].reshape(F, 3, B).transpose(2, 0, 1)
    return alphas


# speedup vs baseline: 61.8408x; 1.0691x over previous
import functools

import jax
import jax.numpy as jnp
from jax import lax
from jax.experimental import pallas as pl
from jax.experimental.pallas import tpu as pltpu
from jax.experimental.pallas import tpu_sc as plsc

NC = 2
NS = 16
L = 16
NW = NC * NS
BLK = 128

_PI = 3.14159265358979


def _rsqrt(x):
    i = lax.bitcast_convert_type(x, jnp.int32)
    i = jnp.int32(0x5F3759DF) - jnp.right_shift(i, 1)
    y = lax.bitcast_convert_type(i, jnp.float32)
    xh = 0.5 * x
    y = y * (1.5 - xh * y * y)
    y = y * (1.5 - xh * y * y)
    return y


def _acos(x):
    ax = jnp.abs(x)
    u = 1.0 - ax
    s = u * _rsqrt(jnp.maximum(u, 1e-30))
    p = jnp.float32(-0.0012624911)
    p = p * ax + 0.0066700901
    p = p * ax + -0.0170881256
    p = p * ax + 0.0308918810
    p = p * ax + -0.0501743046
    p = p * ax + 0.0889789874
    p = p * ax + -0.2145988016
    p = p * ax + 1.5707963050
    r = s * p
    return jnp.where(x < 0, _PI - r, r)


def _build_sc_call(V, F_PAD, D, FPW, NBLK):
    mesh = plsc.VectorSubcoreMesh(core_axis_name="c", subcore_axis_name="s")
    IPW = FPW + BLK

    @functools.partial(
        pl.kernel,
        out_type=jax.ShapeDtypeStruct((F_PAD, D), jnp.float32),
        mesh=mesh,
        compiler_params=pltpu.CompilerParams(use_tc_tiling_on_sc=False),
        scratch_types=[
            pltpu.VMEM((IPW,), jnp.int32),
            pltpu.VMEM((IPW,), jnp.int32),
            pltpu.VMEM((IPW,), jnp.int32),
            pltpu.VMEM((BLK, D), jnp.float32),
            pltpu.VMEM((BLK, D), jnp.float32),
            pltpu.VMEM((BLK, D), jnp.float32),
            pltpu.VMEM((BLK, D), jnp.float32),
            pltpu.VMEM((BLK, D), jnp.float32),
            pltpu.VMEM((BLK, D), jnp.float32),
            pltpu.VMEM((BLK, D), jnp.float32),
            pltpu.VMEM((BLK, D), jnp.float32),
            pltpu.SemaphoreType.DMA,
            pltpu.SemaphoreType.DMA,
            pltpu.SemaphoreType.DMA,
            pltpu.SemaphoreType.DMA,
        ],
    )
    def sc_angles(fs_hbm, i0_hbm, i1_hbm, i2_hbm, out_hbm,
                  i0_v, i1_v, i2_v,
                  p0a, p1a, p2a, p0b, p1b, p2b, oa, ob,
                  sga, sgb, soa, sob):
        wid = lax.axis_index("s") * NC + lax.axis_index("c")
        base = wid * FPW

        pltpu.sync_copy(i0_hbm.at[pl.ds(base, IPW)], i0_v)
        pltpu.sync_copy(i1_hbm.at[pl.ds(base, IPW)], i1_v)
        pltpu.sync_copy(i2_hbm.at[pl.ds(base, IPW)], i2_v)

        def prefetch(blk, p0, p1, p2, sem):
            o = blk * BLK
            pltpu.async_copy(fs_hbm.at[i0_v.at[pl.ds(o, BLK)]], p0, sem)
            pltpu.async_copy(fs_hbm.at[i1_v.at[pl.ds(o, BLK)]], p1, sem)
            pltpu.async_copy(fs_hbm.at[i2_v.at[pl.ds(o, BLK)]], p2, sem)

        def wait_gathers(p0, p1, p2, sem):
            pltpu.make_async_copy(fs_hbm.at[i0_v.at[pl.ds(0, BLK)]], p0, sem).wait()
            pltpu.make_async_copy(fs_hbm.at[i1_v.at[pl.ds(0, BLK)]], p1, sem).wait()
            pltpu.make_async_copy(fs_hbm.at[i2_v.at[pl.ds(0, BLK)]], p2, sem).wait()

        def compute(p0_v, p1_v, p2_v, o_v):
            @pl.loop(0, BLK)
            def _(f):
                p0x = p0_v[f, pl.ds(0, L)]
                p0y = p0_v[f, pl.ds(L, L)]
                p0z = p0_v[f, pl.ds(2 * L, L)]
                p1x = p1_v[f, pl.ds(0, L)]
                p1y = p1_v[f, pl.ds(L, L)]
                p1z = p1_v[f, pl.ds(2 * L, L)]
                p2x = p2_v[f, pl.ds(0, L)]
                p2y = p2_v[f, pl.ds(L, L)]
                p2z = p2_v[f, pl.ds(2 * L, L)]
                e0x, e0y, e0z = p2x - p0x, p2y - p0y, p2z - p0z
                e1x, e1y, e1z = p0x - p1x, p0y - p1y, p0z - p1z
                e2x, e2y, e2z = p1x - p2x, p1y - p2y, p1z - p2z
                n0 = e0x * e0x + e0y * e0y + e0z * e0z
                n1 = e1x * e1x + e1y * e1y + e1z * e1z
                n2 = e2x * e2x + e2y * e2y + e2z * e2z
                m12 = e1x * e2x + e1y * e2y + e1z * e2z
                m20 = e2x * e0x + e2y * e0y + e2z * e0z
                m01 = e0x * e1x + e0y * e1y + e0z * e1z
                c0 = -m12 * _rsqrt(n1 * n2)
                c1 = -m20 * _rsqrt(n2 * n0)
                c2 = -m01 * _rsqrt(n0 * n1)
                one = jnp.float32(1.0)
                c0 = jnp.minimum(jnp.maximum(c0, -one), one)
                c1 = jnp.minimum(jnp.maximum(c1, -one), one)
                c2 = jnp.minimum(jnp.maximum(c2, -one), one)
                o_v[f, pl.ds(0, L)] = _acos(c0)
                o_v[f, pl.ds(L, L)] = _acos(c1)
                o_v[f, pl.ds(2 * L, L)] = _acos(c2)

        def store(blk, o_v, sem):
            pltpu.async_copy(o_v, out_hbm.at[pl.ds(base + blk * BLK, BLK)], sem)

        def wait_store(o_v, sem):
            pltpu.make_async_copy(o_v, out_hbm.at[pl.ds(base, BLK)], sem).wait()

        prefetch(0, p0a, p1a, p2a, sga)

        @pl.loop(0, NBLK // 2)
        def _(i):
            a_blk = 2 * i
            b_blk = 2 * i + 1
            wait_gathers(p0a, p1a, p2a, sga)
            prefetch(b_blk, p0b, p1b, p2b, sgb)

            @pl.when(i > 0)
            def _():
                wait_store(oa, soa)

            compute(p0a, p1a, p2a, oa)
            store(a_blk, oa, soa)

            wait_gathers(p0b, p1b, p2b, sgb)
            prefetch(b_blk + 1, p0a, p1a, p2a, sga)

            @pl.when(i > 0)
            def _():
                wait_store(ob, sob)

            compute(p0b, p1b, p2b, ob)
            store(b_blk, ob, sob)

        wait_gathers(p0a, p1a, p2a, sga)
        wait_store(oa, soa)
        wait_store(ob, sob)

    return sc_angles


def kernel(fs, faces):
    B, V, _ = fs.shape
    F = faces.shape[0]
    assert B == L
    D = 3 * B
    NBLK2 = -(-F // (NW * 2 * BLK))
    NBLK = 2 * NBLK2
    FPW = NBLK * BLK
    F_PAD = FPW * NW

    fs_t = fs.transpose(1, 2, 0).reshape(V, D)
    faces_pad = jnp.pad(faces, ((0, F_PAD + BLK - F), (0, 0)))
    i0 = faces_pad[:, 0]
    i1 = faces_pad[:, 1]
    i2 = faces_pad[:, 2]

    out = _build_sc_call(V, F_PAD, D, FPW, NBLK)(fs_t, i0, i1, i2)
    alphas = out[:F].reshape(F, 3, B).transpose(2, 0, 1)
    return alphas


# speedup vs baseline: 62.9451x; 1.0179x over previous
import functools

import jax
import jax.numpy as jnp
from jax import lax
from jax.experimental import pallas as pl
from jax.experimental.pallas import tpu as pltpu
from jax.experimental.pallas import tpu_sc as plsc

NC = 2
NS = 16
L = 16
NW = NC * NS
BLK = 128

_PI = 3.14159265358979


def _rsqrt(x, iters):
    i = lax.bitcast_convert_type(x, jnp.int32)
    i = jnp.int32(0x5F3759DF) - jnp.right_shift(i, 1)
    y = lax.bitcast_convert_type(i, jnp.float32)
    xh = 0.5 * x
    for _ in range(iters):
        y = y * (1.5 - xh * y * y)
    return y


def _acos(x):
    ax = jnp.abs(x)
    u = 1.0 - ax
    s = u * _rsqrt(jnp.maximum(u, 1e-30), 1)
    p = jnp.float32(-0.0187293)
    p = p * ax + 0.0742610
    p = p * ax + -0.2121144
    p = p * ax + 1.5707288
    r = s * p
    return jnp.where(x < 0, _PI - r, r)


def _build_sc_call(V, F_PAD, D, FPW, NBLK):
    mesh = plsc.VectorSubcoreMesh(core_axis_name="c", subcore_axis_name="s")
    IPW = FPW + BLK

    @functools.partial(
        pl.kernel,
        out_type=jax.ShapeDtypeStruct((F_PAD, D), jnp.float32),
        mesh=mesh,
        compiler_params=pltpu.CompilerParams(use_tc_tiling_on_sc=False),
        scratch_types=[
            pltpu.VMEM((IPW,), jnp.int32),
            pltpu.VMEM((IPW,), jnp.int32),
            pltpu.VMEM((IPW,), jnp.int32),
            pltpu.VMEM((BLK, D), jnp.float32),
            pltpu.VMEM((BLK, D), jnp.float32),
            pltpu.VMEM((BLK, D), jnp.float32),
            pltpu.VMEM((BLK, D), jnp.float32),
            pltpu.VMEM((BLK, D), jnp.float32),
            pltpu.VMEM((BLK, D), jnp.float32),
            pltpu.VMEM((BLK, D), jnp.float32),
            pltpu.VMEM((BLK, D), jnp.float32),
            pltpu.SemaphoreType.DMA,
            pltpu.SemaphoreType.DMA,
            pltpu.SemaphoreType.DMA,
            pltpu.SemaphoreType.DMA,
        ],
    )
    def sc_angles(fs_hbm, i0_hbm, i1_hbm, i2_hbm, out_hbm,
                  i0_v, i1_v, i2_v,
                  p0a, p1a, p2a, p0b, p1b, p2b, oa, ob,
                  sga, sgb, soa, sob):
        wid = lax.axis_index("s") * NC + lax.axis_index("c")
        base = wid * FPW

        pltpu.sync_copy(i0_hbm.at[pl.ds(base, IPW)], i0_v)
        pltpu.sync_copy(i1_hbm.at[pl.ds(base, IPW)], i1_v)
        pltpu.sync_copy(i2_hbm.at[pl.ds(base, IPW)], i2_v)

        def prefetch(blk, p0, p1, p2, sem):
            o = blk * BLK
            pltpu.async_copy(fs_hbm.at[i0_v.at[pl.ds(o, BLK)]], p0, sem)
            pltpu.async_copy(fs_hbm.at[i1_v.at[pl.ds(o, BLK)]], p1, sem)
            pltpu.async_copy(fs_hbm.at[i2_v.at[pl.ds(o, BLK)]], p2, sem)

        def wait_gathers(p0, p1, p2, sem):
            pltpu.make_async_copy(fs_hbm.at[i0_v.at[pl.ds(0, BLK)]], p0, sem).wait()
            pltpu.make_async_copy(fs_hbm.at[i1_v.at[pl.ds(0, BLK)]], p1, sem).wait()
            pltpu.make_async_copy(fs_hbm.at[i2_v.at[pl.ds(0, BLK)]], p2, sem).wait()

        def compute(p0_v, p1_v, p2_v, o_v):
            @pl.loop(0, BLK)
            def _(f):
                p0x = p0_v[f, pl.ds(0, L)]
                p0y = p0_v[f, pl.ds(L, L)]
                p0z = p0_v[f, pl.ds(2 * L, L)]
                p1x = p1_v[f, pl.ds(0, L)]
                p1y = p1_v[f, pl.ds(L, L)]
                p1z = p1_v[f, pl.ds(2 * L, L)]
                p2x = p2_v[f, pl.ds(0, L)]
                p2y = p2_v[f, pl.ds(L, L)]
                p2z = p2_v[f, pl.ds(2 * L, L)]
                e0x, e0y, e0z = p2x - p0x, p2y - p0y, p2z - p0z
                e1x, e1y, e1z = p0x - p1x, p0y - p1y, p0z - p1z
                n0 = e0x * e0x + e0y * e0y + e0z * e0z
                n1 = e1x * e1x + e1y * e1y + e1z * e1z
                g01 = e0x * e1x + e0y * e1y + e0z * e1z
                n2 = n0 + n1 + (g01 + g01)
                c0 = (n1 + g01) * _rsqrt(n1 * n2, 2)
                c1 = (n0 + g01) * _rsqrt(n2 * n0, 2)
                c2 = -g01 * _rsqrt(n0 * n1, 2)
                one = jnp.float32(1.0)
                c0 = jnp.minimum(jnp.maximum(c0, -one), one)
                c1 = jnp.minimum(jnp.maximum(c1, -one), one)
                c2 = jnp.minimum(jnp.maximum(c2, -one), one)
                o_v[f, pl.ds(0, L)] = _acos(c0)
                o_v[f, pl.ds(L, L)] = _acos(c1)
                o_v[f, pl.ds(2 * L, L)] = _acos(c2)

        def store(blk, o_v, sem):
            pltpu.async_copy(o_v, out_hbm.at[pl.ds(base + blk * BLK, BLK)], sem)

        def wait_store(o_v, sem):
            pltpu.make_async_copy(o_v, out_hbm.at[pl.ds(base, BLK)], sem).wait()

        prefetch(0, p0a, p1a, p2a, sga)

        @pl.loop(0, NBLK // 2)
        def _(i):
            a_blk = 2 * i
            b_blk = 2 * i + 1
            wait_gathers(p0a, p1a, p2a, sga)
            prefetch(b_blk, p0b, p1b, p2b, sgb)

            @pl.when(i > 0)
            def _():
                wait_store(oa, soa)

            compute(p0a, p1a, p2a, oa)
            store(a_blk, oa, soa)

            wait_gathers(p0b, p1b, p2b, sgb)
            prefetch(b_blk + 1, p0a, p1a, p2a, sga)

            @pl.when(i > 0)
            def _():
                wait_store(ob, sob)

            compute(p0b, p1b, p2b, ob)
            store(b_blk, ob, sob)

        wait_gathers(p0a, p1a, p2a, sga)
        wait_store(oa, soa)
        wait_store(ob, sob)

    return sc_angles


def kernel(fs, faces):
    B, V, _ = fs.shape
    F = faces.shape[0]
    assert B == L
    D = 3 * B
    NBLK2 = -(-F // (NW * 2 * BLK))
    NBLK = 2 * NBLK2
    FPW = NBLK * BLK
    F_PAD = FPW * NW

    fs_t = fs.transpose(1, 2, 0).reshape(V, D)
    faces_pad = jnp.pad(faces, ((0, F_PAD + BLK - F), (0, 0)))
    i0 = faces_pad[:, 0]
    i1 = faces_pad[:, 1]
    i2 = faces_pad[:, 2]

    out = _build_sc_call(V, F_PAD, D, FPW, NBLK)(fs_t, i0, i1, i2)
    alphas = out[:F].reshape(F, 3, B).transpose(2, 0, 1)
    return alphas
